# baseline (speedup 1.0000x reference)
import jax
import jax.numpy as jnp
from jax import lax
from jax.experimental import pallas as pl
from jax.experimental.pallas import tpu as pltpu

_N = 2048
_B = 4
_TILE = 256
_GRID = _N // _TILE
_BF = jnp.bfloat16


def _bdiag(blocks):
    rows = []
    for i, bi in enumerate(blocks):
        pieces = []
        for j, bj in enumerate(blocks):
            pieces.append(bi if i == j else
                          jnp.zeros((bi.shape[0], bj.shape[1]), bi.dtype))
        rows.append(jnp.concatenate(pieces, axis=1))
    return jnp.concatenate(rows, axis=0)


def _kron4(m):
    return _bdiag([m, m, m, m])


def _tile4(row):
    return jnp.concatenate([row, row, row, row], axis=1)


def _tridiag(wc_ref, g):
    c = wc_ref[0]
    row = lax.broadcasted_iota(jnp.int32, (g, g), 0)
    col = lax.broadcasted_iota(jnp.int32, (g, g), 1)
    z = jnp.zeros((g, g), jnp.float32)
    t = (jnp.where(col == row, c[:, 1:2], z)
         + jnp.where(col == row + 1, c[:, 0:1], z)
         + jnp.where(col == row - 1, c[:, 2:3], z))
    return t


def _crow(bc_ref, g):
    return jnp.broadcast_to(bc_ref[...].reshape(1, 1), (1, g))


def _mega_body(a_ref, xh_ref, xd_ref, xw_ref,
               wgh1_ref, bgh1_ref, wch1_ref, bch1_ref,
               wgd1_ref, bgd1_ref, wcd1_ref, bcd1_ref,
               wgw1_ref, bgw1_ref, wcw1_ref, bcw1_ref,
               wgh2_ref, bgh2_ref, wch2_ref, bch2_ref,
               wgd2_ref, bgd2_ref, wcd2_ref, bcd2_ref,
               wgw2_ref, bgw2_ref, wcw2_ref, bcw2_ref,
               wld_ref, bld_ref, wh_ref, wd_ref, ww_ref,
               out_ref, ls_ref):
    f32 = jnp.float32

    row = lax.broadcasted_iota(jnp.int32, (_TILE, _N), 0)
    col = lax.broadcasted_iota(jnp.int32, (_TILE, _N), 1)
    dparts = []
    for i in range(_GRID):
        sl = pl.ds(i * _TILE, _TILE)
        at = a_ref[sl, :]
        dmask = col == row + i * _TILE
        diag = jnp.sum(jnp.where(dmask, at, 0.0), axis=1)
        deg = jnp.sum(at, axis=1) - diag
        pos = deg > 0.0
        dv = jnp.where(pos, lax.rsqrt(jnp.where(pos, deg, 1.0)),
                       0.0)[:, None]
        dparts.append(dv)
        ls_ref[sl, :] = jnp.where(dmask, 0.0, -dv * at).astype(_BF)
    dinv = jnp.concatenate(dparts, axis=0)

    def hop(x):
        return jnp.dot(ls_ref[...], (dinv * x).astype(_BF),
                       preferred_element_type=f32)

    zpad = jnp.zeros((_N, 4), f32)
    pieces = []
    for b in range(_B):
        pieces += [xh_ref[b], xd_ref[b], xw_ref[b], zpad]
    xc = jnp.concatenate(pieces, axis=1)
    xcbf = xc.astype(_BF)

    def m1(k):
        z = jnp.zeros
        top = jnp.concatenate([wgh1_ref[k], z((24, 128), f32)], axis=1)
        mid = jnp.concatenate([z((12, 64), f32), wgd1_ref[k],
                               z((12, 64), f32)], axis=1)
        bot = jnp.concatenate([z((24, 128), f32), wgw1_ref[k]], axis=1)
        return jnp.concatenate([top, mid, bot, z((4, 192), f32)], axis=0)

    b1w = [_kron4(m1(k)).astype(_BF) for k in range(3)]
    b1row = _tile4(jnp.concatenate(
        [bgh1_ref[...].reshape(1, 64), bgd1_ref[...].reshape(1, 64),
         bgw1_ref[...].reshape(1, 64)], axis=1))
    t1 = _kron4(_bdiag([_tridiag(wch1_ref, 64), _tridiag(wcd1_ref, 64),
                        _tridiag(wcw1_ref, 64)])).astype(_BF)
    c1row = _tile4(jnp.concatenate(
        [_crow(bch1_ref, 64), _crow(bcd1_ref, 64), _crow(bcw1_ref, 64)],
        axis=1))

    b2w = [_kron4(_bdiag([wgh2_ref[k], wgd2_ref[k], wgw2_ref[k]])
                  ).astype(_BF) for k in range(3)]
    b2row = _tile4(jnp.concatenate(
        [bgh2_ref[...].reshape(1, 32), bgd2_ref[...].reshape(1, 32),
         bgw2_ref[...].reshape(1, 32)], axis=1))
    t2 = _kron4(_bdiag([_tridiag(wch2_ref, 32), _tridiag(wcd2_ref, 32),
                        _tridiag(wcw2_ref, 32)])).astype(_BF)
    c2row = _tile4(jnp.concatenate(
        [_crow(bch2_ref, 32), _crow(bcd2_ref, 32), _crow(bcw2_ref, 32)],
        axis=1))

    wldt = jnp.transpose(wld_ref[...])
    l3 = _kron4(_bdiag([wldt, wldt, wldt])).astype(_BF)
    l3row = _tile4(jnp.concatenate(
        [bld_ref[...].reshape(1, 12)] * 3, axis=1))

    def sdiag(s_ref):
        r = lax.broadcasted_iota(jnp.int32, (12, 12), 0)
        c = lax.broadcasted_iota(jnp.int32, (12, 12), 1)
        v = jnp.broadcast_to(s_ref[...].reshape(1, 12), (12, 12))
        return jnp.where(r == c, v, 0.0)

    scomb = jnp.concatenate(
        [sdiag(wh_ref), sdiag(wd_ref), sdiag(ww_ref)], axis=0)
    zc = jnp.zeros((36, 12), jnp.float32)
    combs = [jnp.concatenate(
        [scomb if bb == b else zc for bb in range(_B)],
        axis=0).astype(_BF) for b in range(_B)]

    def cheb_stage(tx0, tx1, tx2, ws, brow, t, crow):
        o = (jnp.dot(tx0, ws[0], preferred_element_type=f32)
             + jnp.dot(tx1, ws[1], preferred_element_type=f32)
             + jnp.dot(tx2, ws[2], preferred_element_type=f32))
        o = jnp.maximum(o + brow, 0.0).astype(_BF)
        o = jnp.dot(o, t, preferred_element_type=f32)
        return jnp.maximum(o + crow, 0.0).astype(_BF)

    tx1 = hop(xcbf).astype(_BF)
    tx2 = (2.0 * hop(tx1) - xc).astype(_BF)
    y1 = cheb_stage(xcbf, tx1, tx2, b1w, b1row, t1, c1row)

    z0 = jnp.dot(y1, b2w[0], preferred_element_type=f32)
    z1 = jnp.dot(y1, b2w[1], preferred_element_type=f32)
    z2 = jnp.dot(y1, b2w[2], preferred_element_type=f32)
    o = z0 - z2 + hop(z1 + 2.0 * hop(z2))
    o = jnp.maximum(o + b2row, 0.0).astype(_BF)
    o = jnp.dot(o, t2, preferred_element_type=f32)
    y2 = jnp.maximum(o + c2row, 0.0).astype(_BF)

    p = jnp.maximum(jnp.dot(y2, l3, preferred_element_type=f32)
                    + l3row, 0.0).astype(_BF)
    for b in range(_B):
        out_ref[b] = jnp.dot(p, combs[b], preferred_element_type=f32)


def kernel(Xh, Xd, Xw, A, WgH1, bgH1, wcH1, bcH1, WgH2, bgH2, wcH2, bcH2,
           WgD1, bgD1, wcD1, bcD1, WgD2, bgD2, wcD2, bcD2,
           WgW1, bgW1, wcW1, bcW1, WgW2, bgW2, wcW2, bcW2,
           WlD, blD, Wh, Wd, Ww):
    out = pl.pallas_call(
        _mega_body,
        out_shape=jax.ShapeDtypeStruct((_B, _N, 12), jnp.float32),
        scratch_shapes=[pltpu.VMEM((_N, _N), _BF)],
    )(A,
      Xh.reshape(_B, _N, 24), Xd.reshape(_B, _N, 12), Xw.reshape(_B, _N, 24),
      WgH1, bgH1, wcH1, bcH1, WgD1, bgD1, wcD1, bcD1,
      WgW1, bgW1, wcW1, bcW1,
      WgH2, bgH2, wcH2, bcH2, WgD2, bgD2, wcD2, bcD2,
      WgW2, bgW2, wcW2, bcW2,
      WlD, blD, Wh, Wd, Ww)
    return out[:, :, None, :]


# speedup vs baseline: 1.0018x; 1.0018x over previous
import jax
import jax.numpy as jnp
from jax import lax
from jax.experimental import pallas as pl
from jax.experimental.pallas import tpu as pltpu

_N = 2048
_B = 4
_TILE = 256
_GRID = _N // _TILE
_BF = jnp.bfloat16


def _bdiag(blocks):
    rows = []
    for i, bi in enumerate(blocks):
        pieces = []
        for j, bj in enumerate(blocks):
            pieces.append(bi if i == j else
                          jnp.zeros((bi.shape[0], bj.shape[1]), bi.dtype))
        rows.append(jnp.concatenate(pieces, axis=1))
    return jnp.concatenate(rows, axis=0)


def _kron4(m):
    return _bdiag([m, m, m, m])


def _tile4(row):
    return jnp.concatenate([row, row, row, row], axis=1)


def _tridiag(wc_ref, g):
    c = wc_ref[0]
    row = lax.broadcasted_iota(jnp.int32, (g, g), 0)
    col = lax.broadcasted_iota(jnp.int32, (g, g), 1)
    z = jnp.zeros((g, g), jnp.float32)
    t = (jnp.where(col == row, c[:, 1:2], z)
         + jnp.where(col == row + 1, c[:, 0:1], z)
         + jnp.where(col == row - 1, c[:, 2:3], z))
    return t


def _crow(bc_ref, g):
    return jnp.broadcast_to(bc_ref[...].reshape(1, 1), (1, g))


def _mega_body(a_ref, xh_ref, xd_ref, xw_ref,
               wgh1_ref, bgh1_ref, wch1_ref, bch1_ref,
               wgd1_ref, bgd1_ref, wcd1_ref, bcd1_ref,
               wgw1_ref, bgw1_ref, wcw1_ref, bcw1_ref,
               wgh2_ref, bgh2_ref, wch2_ref, bch2_ref,
               wgd2_ref, bgd2_ref, wcd2_ref, bcd2_ref,
               wgw2_ref, bgw2_ref, wcw2_ref, bcw2_ref,
               wld_ref, bld_ref, wh_ref, wd_ref, ww_ref,
               out_ref, ls_ref):
    f32 = jnp.float32

    eye = (lax.broadcasted_iota(jnp.int32, (_TILE, _TILE), 0)
           == lax.broadcasted_iota(jnp.int32, (_TILE, _TILE), 1))
    dparts = []
    for i in range(_GRID):
        sl = pl.ds(i * _TILE, _TILE)
        at = a_ref[sl, :]
        dblk = at[:, i * _TILE:(i + 1) * _TILE]
        diag = jnp.sum(jnp.where(eye, dblk, 0.0), axis=1)
        deg = jnp.sum(at, axis=1) - diag
        pos = deg > 0.0
        dv = jnp.where(pos, lax.rsqrt(jnp.where(pos, deg, 1.0)),
                       0.0)[:, None]
        dparts.append(dv)
        ls_ref[sl, :] = (-dv * at).astype(_BF)
        ls_ref[sl, i * _TILE:(i + 1) * _TILE] = jnp.where(
            eye, 0.0, (-dv * dblk)).astype(_BF)
    dinv = jnp.concatenate(dparts, axis=0)

    def hop(x):
        return jnp.dot(ls_ref[...], (dinv * x).astype(_BF),
                       preferred_element_type=f32)

    zpad = jnp.zeros((_N, 4), f32)
    pieces = []
    for b in range(_B):
        pieces += [xh_ref[b], xd_ref[b], xw_ref[b], zpad]
    xc = jnp.concatenate(pieces, axis=1)
    xcbf = xc.astype(_BF)

    def m1(k):
        z = jnp.zeros
        top = jnp.concatenate([wgh1_ref[k], z((24, 128), f32)], axis=1)
        mid = jnp.concatenate([z((12, 64), f32), wgd1_ref[k],
                               z((12, 64), f32)], axis=1)
        bot = jnp.concatenate([z((24, 128), f32), wgw1_ref[k]], axis=1)
        return jnp.concatenate([top, mid, bot, z((4, 192), f32)], axis=0)

    b1w = [_kron4(m1(k)).astype(_BF) for k in range(3)]
    b1row = _tile4(jnp.concatenate(
        [bgh1_ref[...].reshape(1, 64), bgd1_ref[...].reshape(1, 64),
         bgw1_ref[...].reshape(1, 64)], axis=1))
    t1 = _kron4(_bdiag([_tridiag(wch1_ref, 64), _tridiag(wcd1_ref, 64),
                        _tridiag(wcw1_ref, 64)])).astype(_BF)
    c1row = _tile4(jnp.concatenate(
        [_crow(bch1_ref, 64), _crow(bcd1_ref, 64), _crow(bcw1_ref, 64)],
        axis=1))

    b2w = [_kron4(_bdiag([wgh2_ref[k], wgd2_ref[k], wgw2_ref[k]])
                  ).astype(_BF) for k in range(3)]
    b2row = _tile4(jnp.concatenate(
        [bgh2_ref[...].reshape(1, 32), bgd2_ref[...].reshape(1, 32),
         bgw2_ref[...].reshape(1, 32)], axis=1))
    t2 = _kron4(_bdiag([_tridiag(wch2_ref, 32), _tridiag(wcd2_ref, 32),
                        _tridiag(wcw2_ref, 32)])).astype(_BF)
    c2row = _tile4(jnp.concatenate(
        [_crow(bch2_ref, 32), _crow(bcd2_ref, 32), _crow(bcw2_ref, 32)],
        axis=1))

    wldt = jnp.transpose(wld_ref[...])
    l3 = _kron4(_bdiag([wldt, wldt, wldt])).astype(_BF)
    l3row = _tile4(jnp.concatenate(
        [bld_ref[...].reshape(1, 12)] * 3, axis=1))

    def sdiag(s_ref):
        r = lax.broadcasted_iota(jnp.int32, (12, 12), 0)
        c = lax.broadcasted_iota(jnp.int32, (12, 12), 1)
        v = jnp.broadcast_to(s_ref[...].reshape(1, 12), (12, 12))
        return jnp.where(r == c, v, 0.0)

    scomb = jnp.concatenate(
        [sdiag(wh_ref), sdiag(wd_ref), sdiag(ww_ref)], axis=0)
    zc = jnp.zeros((36, 12), jnp.float32)
    combs = [jnp.concatenate(
        [scomb if bb == b else zc for bb in range(_B)],
        axis=0).astype(_BF) for b in range(_B)]

    def cheb_stage(tx0, tx1, tx2, ws, brow, t, crow):
        o = (jnp.dot(tx0, ws[0], preferred_element_type=f32)
             + jnp.dot(tx1, ws[1], preferred_element_type=f32)
             + jnp.dot(tx2, ws[2], preferred_element_type=f32))
        o = jnp.maximum(o + brow, 0.0).astype(_BF)
        o = jnp.dot(o, t, preferred_element_type=f32)
        return jnp.maximum(o + crow, 0.0).astype(_BF)

    tx1 = hop(xcbf).astype(_BF)
    tx2 = (2.0 * hop(tx1) - xc).astype(_BF)
    y1 = cheb_stage(xcbf, tx1, tx2, b1w, b1row, t1, c1row)

    z0 = jnp.dot(y1, b2w[0], preferred_element_type=f32)
    z1 = jnp.dot(y1, b2w[1], preferred_element_type=f32)
    z2 = jnp.dot(y1, b2w[2], preferred_element_type=f32)
    o = z0 - z2 + hop(z1 + 2.0 * hop(z2))
    o = jnp.maximum(o + b2row, 0.0).astype(_BF)
    o = jnp.dot(o, t2, preferred_element_type=f32)
    y2 = jnp.maximum(o + c2row, 0.0).astype(_BF)

    p = jnp.maximum(jnp.dot(y2, l3, preferred_element_type=f32)
                    + l3row, 0.0).astype(_BF)
    for b in range(_B):
        out_ref[b] = jnp.dot(p, combs[b], preferred_element_type=f32)


def kernel(Xh, Xd, Xw, A, WgH1, bgH1, wcH1, bcH1, WgH2, bgH2, wcH2, bcH2,
           WgD1, bgD1, wcD1, bcD1, WgD2, bgD2, wcD2, bcD2,
           WgW1, bgW1, wcW1, bcW1, WgW2, bgW2, wcW2, bcW2,
           WlD, blD, Wh, Wd, Ww):
    out = pl.pallas_call(
        _mega_body,
        out_shape=jax.ShapeDtypeStruct((_B, _N, 12), jnp.float32),
        scratch_shapes=[pltpu.VMEM((_N, _N), _BF)],
    )(A,
      Xh.reshape(_B, _N, 24), Xd.reshape(_B, _N, 12), Xw.reshape(_B, _N, 24),
      WgH1, bgH1, wcH1, bcH1, WgD1, bgD1, wcD1, bcD1,
      WgW1, bgW1, wcW1, bcW1,
      WgH2, bgH2, wcH2, bcH2, WgD2, bgD2, wcD2, bcD2,
      WgW2, bgW2, wcW2, bcW2,
      WlD, blD, Wh, Wd, Ww)
    return out[:, :, None, :]


# speedup vs baseline: 1.0159x; 1.0141x over previous
import jax
import jax.numpy as jnp
from jax import lax
from jax.experimental import pallas as pl
from jax.experimental.pallas import tpu as pltpu

_N = 2048
_B = 4
_TILE = 256
_GRID = _N // _TILE
_BF = jnp.bfloat16


def _bdiag(blocks):
    rows = []
    for i, bi in enumerate(blocks):
        pieces = []
        for j, bj in enumerate(blocks):
            pieces.append(bi if i == j else
                          jnp.zeros((bi.shape[0], bj.shape[1]), bi.dtype))
        rows.append(jnp.concatenate(pieces, axis=1))
    return jnp.concatenate(rows, axis=0)


def _kron4(m):
    return _bdiag([m, m, m, m])


def _tile4(row):
    return jnp.concatenate([row, row, row, row], axis=1)


def _tridiag(wc_ref, g):
    c = wc_ref[0]
    row = lax.broadcasted_iota(jnp.int32, (g, g), 0)
    col = lax.broadcasted_iota(jnp.int32, (g, g), 1)
    z = jnp.zeros((g, g), jnp.float32)
    t = (jnp.where(col == row, c[:, 1:2], z)
         + jnp.where(col == row + 1, c[:, 0:1], z)
         + jnp.where(col == row - 1, c[:, 2:3], z))
    return t


def _crow(bc_ref, g):
    return jnp.broadcast_to(bc_ref[...].reshape(1, 1), (1, g))


def _mega_body(a_ref, xh_ref, xd_ref, xw_ref,
               wgh1_ref, bgh1_ref, wch1_ref, bch1_ref,
               wgd1_ref, bgd1_ref, wcd1_ref, bcd1_ref,
               wgw1_ref, bgw1_ref, wcw1_ref, bcw1_ref,
               wgh2_ref, bgh2_ref, wch2_ref, bch2_ref,
               wgd2_ref, bgd2_ref, wcd2_ref, bcd2_ref,
               wgw2_ref, bgw2_ref, wcw2_ref, bcw2_ref,
               wld_ref, bld_ref, wh_ref, wd_ref, ww_ref,
               out_ref, ls_ref):
    f32 = jnp.float32

    eye = (lax.broadcasted_iota(jnp.int32, (_TILE, _TILE), 0)
           == lax.broadcasted_iota(jnp.int32, (_TILE, _TILE), 1))
    dparts = []
    for i in range(_GRID):
        sl = pl.ds(i * _TILE, _TILE)
        at = a_ref[sl, :]
        dblk = at[:, i * _TILE:(i + 1) * _TILE]
        diag = jnp.sum(jnp.where(eye, dblk, 0.0), axis=1)
        deg = jnp.sum(at, axis=1) - diag
        pos = deg > 0.0
        dv = jnp.where(pos, lax.rsqrt(jnp.where(pos, deg, 1.0)),
                       0.0)[:, None]
        dparts.append(dv)
        ls_ref[sl, :] = (-dv * at).astype(_BF)
        ls_ref[sl, i * _TILE:(i + 1) * _TILE] = jnp.where(
            eye, 0.0, (-dv * dblk)).astype(_BF)
    dinv = jnp.concatenate(dparts, axis=0)

    def hop(x):
        return jnp.dot(ls_ref[...], (dinv * x).astype(_BF),
                       preferred_element_type=f32)

    zpad = jnp.zeros((_N, 4), f32)
    pieces = []
    for b in range(_B):
        pieces += [xh_ref[b], xd_ref[b], xw_ref[b], zpad]
    xc = jnp.concatenate(pieces, axis=1)
    xcbf = xc.astype(_BF)

    def m1(k):
        z = jnp.zeros
        top = jnp.concatenate([wgh1_ref[k], z((24, 128), f32)], axis=1)
        mid = jnp.concatenate([z((12, 64), f32), wgd1_ref[k],
                               z((12, 64), f32)], axis=1)
        bot = jnp.concatenate([z((24, 128), f32), wgw1_ref[k]], axis=1)
        return jnp.concatenate([top, mid, bot, z((4, 192), f32)], axis=0)

    b1w = [_kron4(m1(k)).astype(_BF) for k in range(3)]
    b1row = _tile4(jnp.concatenate(
        [bgh1_ref[...].reshape(1, 64), bgd1_ref[...].reshape(1, 64),
         bgw1_ref[...].reshape(1, 64)], axis=1))
    t1 = _kron4(_bdiag([_tridiag(wch1_ref, 64), _tridiag(wcd1_ref, 64),
                        _tridiag(wcw1_ref, 64)])).astype(_BF)
    c1row = _tile4(jnp.concatenate(
        [_crow(bch1_ref, 64), _crow(bcd1_ref, 64), _crow(bcw1_ref, 64)],
        axis=1))

    b2w = [_kron4(_bdiag([wgh2_ref[k], wgd2_ref[k], wgw2_ref[k]])
                  ).astype(_BF) for k in range(3)]
    b2row = _tile4(jnp.concatenate(
        [bgh2_ref[...].reshape(1, 32), bgd2_ref[...].reshape(1, 32),
         bgw2_ref[...].reshape(1, 32)], axis=1))
    t2 = _kron4(_bdiag([_tridiag(wch2_ref, 32), _tridiag(wcd2_ref, 32),
                        _tridiag(wcw2_ref, 32)])).astype(_BF)
    c2row = _tile4(jnp.concatenate(
        [_crow(bch2_ref, 32), _crow(bcd2_ref, 32), _crow(bcw2_ref, 32)],
        axis=1))

    wldt = jnp.transpose(wld_ref[...])
    l3 = _kron4(_bdiag([wldt, wldt, wldt])).astype(_BF)
    l3row = _tile4(jnp.concatenate(
        [bld_ref[...].reshape(1, 12)] * 3, axis=1))

    def sdiag(s_ref):
        r = lax.broadcasted_iota(jnp.int32, (12, 12), 0)
        c = lax.broadcasted_iota(jnp.int32, (12, 12), 1)
        v = jnp.broadcast_to(s_ref[...].reshape(1, 12), (12, 12))
        return jnp.where(r == c, v, 0.0)

    comb = _kron4(jnp.concatenate(
        [sdiag(wh_ref), sdiag(wd_ref), sdiag(ww_ref)], axis=0))

    def cheb_stage(tx0, tx1, tx2, ws, brow, t, crow):
        o = (jnp.dot(tx0, ws[0], preferred_element_type=f32)
             + jnp.dot(tx1, ws[1], preferred_element_type=f32)
             + jnp.dot(tx2, ws[2], preferred_element_type=f32))
        o = jnp.maximum(o + brow, 0.0).astype(_BF)
        o = jnp.dot(o, t, preferred_element_type=f32)
        return jnp.maximum(o + crow, 0.0).astype(_BF)

    tx1 = hop(xcbf).astype(_BF)
    tx2 = (2.0 * hop(tx1) - xc).astype(_BF)
    y1 = cheb_stage(xcbf, tx1, tx2, b1w, b1row, t1, c1row)

    z0 = jnp.dot(y1, b2w[0], preferred_element_type=f32)
    z1 = jnp.dot(y1, b2w[1], preferred_element_type=f32)
    z2 = jnp.dot(y1, b2w[2], preferred_element_type=f32)
    o = z0 - z2 + hop(z1 + 2.0 * hop(z2))
    o = jnp.maximum(o + b2row, 0.0).astype(_BF)
    o = jnp.dot(o, t2, preferred_element_type=f32)
    y2 = jnp.maximum(o + c2row, 0.0).astype(_BF)

    p = jnp.maximum(jnp.dot(y2, l3, preferred_element_type=f32)
                    + l3row, 0.0)
    res = jnp.dot(p, comb, preferred_element_type=f32)
    for b in range(_B):
        out_ref[b] = res[:, 12 * b:12 * (b + 1)]


def kernel(Xh, Xd, Xw, A, WgH1, bgH1, wcH1, bcH1, WgH2, bgH2, wcH2, bcH2,
           WgD1, bgD1, wcD1, bcD1, WgD2, bgD2, wcD2, bcD2,
           WgW1, bgW1, wcW1, bcW1, WgW2, bgW2, wcW2, bcW2,
           WlD, blD, Wh, Wd, Ww):
    out = pl.pallas_call(
        _mega_body,
        out_shape=jax.ShapeDtypeStruct((_B, _N, 12), jnp.float32),
        scratch_shapes=[pltpu.VMEM((_N, _N), _BF)],
    )(A,
      Xh.reshape(_B, _N, 24), Xd.reshape(_B, _N, 12), Xw.reshape(_B, _N, 24),
      WgH1, bgH1, wcH1, bcH1, WgD1, bgD1, wcD1, bcD1,
      WgW1, bgW1, wcW1, bcW1,
      WgH2, bgH2, wcH2, bcH2, WgD2, bgD2, wcD2, bcD2,
      WgW2, bgW2, wcW2, bcW2,
      WlD, blD, Wh, Wd, Ww)
    return out[:, :, None, :]
